# Initial kernel scaffold; baseline (speedup 1.0000x reference)
import jax
import jax.numpy as jnp
from jax.experimental import pallas as pl


def kernel(dynamic_emb, emb_rel, W_ih, b_ih, W_hh, b_hh, W_neigh, W_self, time_gate_weight, time_gate_bias, fc_W, fc_b, proj_W, proj_b, twin_h, edge_index, edge_type):
    raise NotImplementedError("write your pallas kernel here")


# speedup vs baseline: 3.0705x; 3.0705x over previous
import functools

import jax
import jax.numpy as jnp
from jax import lax
from jax.experimental import pallas as pl
from jax.experimental.pallas import tpu as pltpu
from jax.experimental.pallas import tpu_sc as plsc

N = 10000
E = 320000
D = 128
NR2 = 400
DW = 144

NC = 2
NS = 16
NW = NC * NS
B = 128
EPT = -(-E // NW)
CH = 8
JB = CH * (-(-EPT // (B * CH)))
EPTP = JB * B
EP = NW * EPTP

R = 10240
RROWS = R // NS
RREL = 416
RRROWS = RREL // NS

_SLOPE = 11.0 / 48.0


def _l2norm(x):
    n = jnp.sqrt(jnp.sum(x * x, axis=-1, keepdims=True))
    return x / jnp.maximum(n, 1e-12)


def _tc0_body(x_ref, o_ref):
    x = x_ref[...]
    hn = _l2norm(x)
    rows = x.shape[0]
    ones = jnp.ones((rows, 1), jnp.float32)
    zeros = jnp.zeros((rows, DW - D - 1), jnp.float32)
    o_ref[...] = jnp.concatenate([hn, ones, zeros], axis=1)


def _tc0(dynamic_emb):
    blk = 400
    return pl.pallas_call(
        _tc0_body,
        grid=(N // blk,),
        in_specs=[pl.BlockSpec((blk, D), lambda i: (i, 0))],
        out_specs=pl.BlockSpec((blk, DW), lambda i: (i, 0)),
        out_shape=jax.ShapeDtypeStruct((N, DW), jnp.float32),
    )(dynamic_emb)


def _sc1_body(h_hbm, src_hbm, dst_hbm, et_hbm, za_hbm, zr_hbm,
              out_a, out_r, srcv, dstv, etv, rows_v, sem,
              aggh_sh, rels_sh):
    c = lax.axis_index("c")
    s = lax.axis_index("s")
    w = c * NS + s
    pltpu.sync_copy(za_hbm, aggh_sh.at[pl.ds(s * RROWS, RROWS)])
    pltpu.sync_copy(zr_hbm, rels_sh.at[pl.ds(s * RRROWS, RRROWS)])
    plsc.subcore_barrier()

    def chunk(jc, carry):
        pltpu.sync_copy(src_hbm.at[w].at[pl.ds(jc * CH, CH)], srcv)
        pltpu.sync_copy(dst_hbm.at[w].at[pl.ds(jc * CH, CH)], dstv)
        pltpu.sync_copy(et_hbm.at[w].at[pl.ds(jc * CH, CH)], etv)
        for j in range(CH):
            pltpu.async_copy(h_hbm.at[srcv.at[j]], rows_v, sem).wait()
            pltpu.sync_copy(rows_v, aggh_sh.at[dstv.at[j]], add=True)
            pltpu.sync_copy(rows_v, rels_sh.at[etv.at[j]], add=True)
        return carry

    lax.fori_loop(0, JB // CH, chunk, 0)
    plsc.subcore_barrier()
    pltpu.sync_copy(aggh_sh.at[pl.ds(s * RROWS, RROWS)],
                    out_a.at[c].at[pl.ds(s * RROWS, RROWS)])
    pltpu.sync_copy(rels_sh.at[pl.ds(s * RRROWS, RRROWS)],
                    out_r.at[c].at[pl.ds(s * RRROWS, RRROWS)])


def _sc1(h144, srcv, dstv, etv):
    mesh = plsc.VectorSubcoreMesh(core_axis_name="c", subcore_axis_name="s",
                                  num_cores=NC, num_subcores=NS)
    za = jnp.zeros((RROWS, DW), jnp.float32)
    zr = jnp.zeros((RRROWS, DW), jnp.float32)
    f = pl.kernel(
        _sc1_body,
        out_type=(jax.ShapeDtypeStruct((NC, R, DW), jnp.float32),
                  jax.ShapeDtypeStruct((NC, RREL, DW), jnp.float32)),
        mesh=mesh,
        compiler_params=pltpu.CompilerParams(use_tc_tiling_on_sc=False),
        scratch_types=[
            pltpu.VMEM((CH, B), jnp.int32),
            pltpu.VMEM((CH, B), jnp.int32),
            pltpu.VMEM((CH, B), jnp.int32),
            pltpu.VMEM((B, DW), jnp.float32),
            pltpu.SemaphoreType.DMA,
            pltpu.VMEM_SHARED((R, DW), jnp.float32),
            pltpu.VMEM_SHARED((RREL, DW), jnp.float32),
        ],
    )
    return f(h144, srcv, dstv, etv, za, zr)


def _tc1_body(relp_ref, emb_ref, wia_ref, wib_ref, bi_ref, wht_ref, bh_ref,
              o_ref):
    rels = relp_ref[0] + relp_ref[1]
    rel_sum = rels[:, :D]
    rel_cnt = rels[:, D:D + 1]
    x_mean = rel_sum / jnp.maximum(rel_cnt, 1.0)
    emb = emb_ref[...]
    gi = (jnp.dot(emb, wia_ref[...], preferred_element_type=jnp.float32)
          + jnp.dot(x_mean, wib_ref[...], preferred_element_type=jnp.float32)
          + bi_ref[...])
    gh = jnp.dot(emb, wht_ref[...], preferred_element_type=jnp.float32) \
        + bh_ref[...]
    i_r, i_z, i_n = gi[:, :D], gi[:, D:2 * D], gi[:, 2 * D:]
    h_r, h_z, h_n = gh[:, :D], gh[:, D:2 * D], gh[:, 2 * D:]
    r = jax.nn.sigmoid(i_r + h_r)
    z = jax.nn.sigmoid(i_z + h_z)
    n = jnp.tanh(i_n + r * h_n)
    h0 = (1.0 - z) * n + z * emb
    o_ref[...] = _l2norm(h0)


def _tc1(out_r, emb_rel, W_ih, b_ih, W_hh, b_hh):
    emb_p = jnp.zeros((RREL, D), jnp.float32).at[:NR2].set(emb_rel)
    wia = W_ih[:, :D].T
    wib = W_ih[:, D:].T
    wht = W_hh.T
    bi = b_ih.reshape(1, 3 * D)
    bh = b_hh.reshape(1, 3 * D)
    return pl.pallas_call(
        _tc1_body,
        out_shape=jax.ShapeDtypeStruct((RREL, D), jnp.float32),
    )(out_r, emb_p, wia, wib, bi, wht, bh)


def _sc2_body(t_hbm, dst_hbm, et_hbm, z_hbm, out_a, dstv, etv, rows_v, sem,
              agg_sh):
    c = lax.axis_index("c")
    s = lax.axis_index("s")
    w = c * NS + s
    pltpu.sync_copy(z_hbm, agg_sh.at[pl.ds(s * RROWS, RROWS)])
    plsc.subcore_barrier()

    def chunk(jc, carry):
        pltpu.sync_copy(dst_hbm.at[w].at[pl.ds(jc * CH, CH)], dstv)
        pltpu.sync_copy(et_hbm.at[w].at[pl.ds(jc * CH, CH)], etv)
        for j in range(CH):
            pltpu.async_copy(t_hbm.at[etv.at[j]], rows_v, sem).wait()
            pltpu.sync_copy(rows_v, agg_sh.at[dstv.at[j]], add=True)
        return carry

    lax.fori_loop(0, JB // CH, chunk, 0)
    plsc.subcore_barrier()
    pltpu.sync_copy(agg_sh.at[pl.ds(s * RROWS, RROWS)],
                    out_a.at[c].at[pl.ds(s * RROWS, RROWS)])


def _sc2(r_emb_p, dstv, etv):
    mesh = plsc.VectorSubcoreMesh(core_axis_name="c", subcore_axis_name="s",
                                  num_cores=NC, num_subcores=NS)
    z = jnp.zeros((RROWS, D), jnp.float32)
    f = pl.kernel(
        _sc2_body,
        out_type=jax.ShapeDtypeStruct((NC, R, D), jnp.float32),
        mesh=mesh,
        compiler_params=pltpu.CompilerParams(use_tc_tiling_on_sc=False),
        scratch_types=[
            pltpu.VMEM((CH, B), jnp.int32),
            pltpu.VMEM((CH, B), jnp.int32),
            pltpu.VMEM((B, D), jnp.float32),
            pltpu.SemaphoreType.DMA,
            pltpu.VMEM_SHARED((R, D), jnp.float32),
        ],
    )
    return f(r_emb_p, dstv, etv, z)


def _tc2_body(agga_ref, aggr_ref, h_ref, tw_ref, wn_ref, ws_ref,
              tgw_ref, tgb_ref, fca_ref, fcb_ref, fbias_ref,
              pw_ref, pb_ref, o_ref):
    a144 = agga_ref[0] + agga_ref[1]
    aggr = aggr_ref[0] + aggr_ref[1]
    ssum = a144[:, :D] + aggr
    deg = a144[:, D:D + 1]
    agg = ssum / jnp.maximum(deg, 1.0)
    h = h_ref[...]
    cur = (jnp.dot(agg, wn_ref[...], preferred_element_type=jnp.float32)
           + jnp.dot(h, ws_ref[...], preferred_element_type=jnp.float32))
    cur = jnp.where(cur >= 0, cur, _SLOPE * cur)
    cur = _l2norm(cur)
    gate = jax.nn.sigmoid(
        jnp.dot(cur, tgw_ref[...], preferred_element_type=jnp.float32)
        + tgb_ref[...])
    h_new = gate * cur + (1.0 - gate) * h
    tw = tw_ref[...]
    xg = jnp.tanh(
        jnp.dot(h_new, fca_ref[...], preferred_element_type=jnp.float32)
        + jnp.dot(tw, fcb_ref[...], preferred_element_type=jnp.float32)
        + fbias_ref[...])
    g = jax.nn.sigmoid(
        jnp.sum(xg * pw_ref[...], axis=-1, keepdims=True) + pb_ref[...])
    o_ref[...] = (1.0 - g) * h_new + g * tw


def _tc2(out_a, out_g, h, twin_h, W_neigh, W_self, tgw, tgb,
         fc_W, fc_b, proj_W, proj_b):
    blk = 512
    grid = R // blk
    hp = jnp.zeros((R, D), jnp.float32).at[:N].set(h)
    twp = jnp.zeros((R, D), jnp.float32).at[:N].set(twin_h)
    fca = fc_W[:, :D].T
    fcb = fc_W[:, D:].T
    out = pl.pallas_call(
        _tc2_body,
        grid=(grid,),
        in_specs=[
            pl.BlockSpec((NC, blk, DW), lambda i: (0, i, 0)),
            pl.BlockSpec((NC, blk, D), lambda i: (0, i, 0)),
            pl.BlockSpec((blk, D), lambda i: (i, 0)),
            pl.BlockSpec((blk, D), lambda i: (i, 0)),
            pl.BlockSpec((D, D), lambda i: (0, 0)),
            pl.BlockSpec((D, D), lambda i: (0, 0)),
            pl.BlockSpec((D, D), lambda i: (0, 0)),
            pl.BlockSpec((1, D), lambda i: (0, 0)),
            pl.BlockSpec((D, D), lambda i: (0, 0)),
            pl.BlockSpec((D, D), lambda i: (0, 0)),
            pl.BlockSpec((1, D), lambda i: (0, 0)),
            pl.BlockSpec((1, D), lambda i: (0, 0)),
            pl.BlockSpec((1, 1), lambda i: (0, 0)),
        ],
        out_specs=pl.BlockSpec((blk, D), lambda i: (i, 0)),
        out_shape=jax.ShapeDtypeStruct((R, D), jnp.float32),
    )(out_a, out_g, hp, twp, W_neigh, W_self, tgw,
      tgb.reshape(1, D), fca, fcb, fc_b.reshape(1, D),
      proj_W.reshape(1, D), proj_b.reshape(1, 1))
    return out[:N]


def kernel(dynamic_emb, emb_rel, W_ih, b_ih, W_hh, b_hh, W_neigh, W_self,
           time_gate_weight, time_gate_bias, fc_W, fc_b, proj_W, proj_b,
           twin_h, edge_index, edge_type):
    pad = EP - E
    src = jnp.concatenate([edge_index[0], jnp.zeros((pad,), jnp.int32)])
    dst = jnp.concatenate([edge_index[1],
                           jnp.full((pad,), N, jnp.int32)])
    et = jnp.concatenate([edge_type, jnp.full((pad,), NR2, jnp.int32)])
    srcv = src.reshape(NW, JB, B)
    dstv = dst.reshape(NW, JB, B)
    etv = et.reshape(NW, JB, B)

    h144 = _tc0(dynamic_emb)
    out_a, out_r = _sc1(h144, srcv, dstv, etv)
    r_emb_p = _tc1(out_r, emb_rel, W_ih, b_ih, W_hh, b_hh)
    out_g = _sc2(r_emb_p, dstv, etv)
    h = h144[:, :D]
    return _tc2(out_a, out_g, h, twin_h, W_neigh, W_self,
                time_gate_weight, time_gate_bias, fc_W, fc_b,
                proj_W, proj_b)


# speedup vs baseline: 5.3911x; 1.7558x over previous
import jax
import jax.numpy as jnp
from jax import lax
from jax.experimental import pallas as pl
from jax.experimental.pallas import tpu as pltpu
from jax.experimental.pallas import tpu_sc as plsc

N = 10000
E = 320000
D = 128
NR2 = 400
DW = 144

NC = 2
NS = 16
NW = NC * NS
B = 112
CH = 6
JB = 90
EP = NW * JB * B

R = 10240
RROWS = R // NS
RREL = 416
RRROWS = RREL // NS

_SLOPE = 11.0 / 48.0


def _l2norm(x):
    n = jnp.sqrt(jnp.sum(x * x, axis=-1, keepdims=True))
    return x / jnp.maximum(n, 1e-12)


def _tc0_body(x_ref, o_ref):
    x = x_ref[...]
    hn = _l2norm(x)
    rows = x.shape[0]
    ones = jnp.ones((rows, 1), jnp.float32)
    zeros = jnp.zeros((rows, DW - D - 1), jnp.float32)
    o_ref[...] = jnp.concatenate([hn, ones, zeros], axis=1)


def _tc0(dynamic_emb):
    blk = 400
    return pl.pallas_call(
        _tc0_body,
        grid=(N // blk,),
        in_specs=[pl.BlockSpec((blk, D), lambda i: (i, 0))],
        out_specs=pl.BlockSpec((blk, DW), lambda i: (i, 0)),
        out_shape=jax.ShapeDtypeStruct((N, DW), jnp.float32),
    )(dynamic_emb)


def _edge_pipeline(table_hbm, idxsrc_hbm, w, srcv, scat_idx_hbm, scat_idx_v,
                   bufs, semg, scat_sems, scat_dsts):
    def wait_scat():
        for sem, dsh, iv in zip(scat_sems, scat_dsts, scat_idx_v):
            pltpu.make_async_copy(bufs[0], dsh.at[iv.at[0]], sem).wait()

    def chunk(jc, carry):
        @pl.when(jc > 0)
        def _():
            wait_scat()
            wait_scat()
        pltpu.sync_copy(idxsrc_hbm.at[w].at[pl.ds(jc * CH, CH)], srcv)
        for ih, iv in zip(scat_idx_hbm, scat_idx_v):
            pltpu.sync_copy(ih.at[w].at[pl.ds(jc * CH, CH)], iv)
        pltpu.async_copy(table_hbm.at[srcv.at[0]], bufs[0], semg[0])
        for j in range(CH):
            buf = bufs[j % 2]
            if j + 1 < CH:
                if j >= 1:
                    wait_scat()
                pltpu.async_copy(table_hbm.at[srcv.at[j + 1]],
                                 bufs[(j + 1) % 2], semg[(j + 1) % 2])
            pltpu.make_async_copy(table_hbm.at[srcv.at[j]], buf,
                                  semg[j % 2]).wait()
            for sem, dsh, iv in zip(scat_sems, scat_dsts, scat_idx_v):
                pltpu.async_copy(buf, dsh.at[iv.at[j]], sem, add=True)
        return carry

    lax.fori_loop(0, JB // CH, chunk, 0)
    wait_scat()
    wait_scat()


def _sc1_body(h_hbm, src_hbm, dst_hbm, et_hbm, za_hbm, zr_hbm,
              out_a, out_r, srcv, dstv, etv, r0, r1, semg0, semg1,
              sema, semr, aggh_sh, rels_sh):
    c = lax.axis_index("c")
    s = lax.axis_index("s")
    w = c * NS + s
    pltpu.sync_copy(za_hbm, aggh_sh.at[pl.ds(s * RROWS, RROWS)])
    pltpu.sync_copy(zr_hbm, rels_sh.at[pl.ds(s * RRROWS, RRROWS)])
    plsc.subcore_barrier()
    _edge_pipeline(h_hbm, src_hbm, w, srcv,
                   [dst_hbm, et_hbm], [dstv, etv],
                   (r0, r1), (semg0, semg1),
                   [sema, semr], [aggh_sh, rels_sh])
    plsc.subcore_barrier()
    pltpu.sync_copy(aggh_sh.at[pl.ds(s * RROWS, RROWS)],
                    out_a.at[c].at[pl.ds(s * RROWS, RROWS)])
    pltpu.sync_copy(rels_sh.at[pl.ds(s * RRROWS, RRROWS)],
                    out_r.at[c].at[pl.ds(s * RRROWS, RRROWS)])


def _sc1(h144, srcv, dstv, etv):
    mesh = plsc.VectorSubcoreMesh(core_axis_name="c", subcore_axis_name="s",
                                  num_cores=NC, num_subcores=NS)
    za = jnp.zeros((RROWS, DW), jnp.float32)
    zr = jnp.zeros((RRROWS, DW), jnp.float32)
    f = pl.kernel(
        _sc1_body,
        out_type=(jax.ShapeDtypeStruct((NC, R, DW), jnp.float32),
                  jax.ShapeDtypeStruct((NC, RREL, DW), jnp.float32)),
        mesh=mesh,
        compiler_params=pltpu.CompilerParams(use_tc_tiling_on_sc=False),
        scratch_types=[
            pltpu.VMEM((CH, B), jnp.int32),
            pltpu.VMEM((CH, B), jnp.int32),
            pltpu.VMEM((CH, B), jnp.int32),
            pltpu.VMEM((B, DW), jnp.float32),
            pltpu.VMEM((B, DW), jnp.float32),
            pltpu.SemaphoreType.DMA,
            pltpu.SemaphoreType.DMA,
            pltpu.SemaphoreType.DMA,
            pltpu.SemaphoreType.DMA,
            pltpu.VMEM_SHARED((R, DW), jnp.float32),
            pltpu.VMEM_SHARED((RREL, DW), jnp.float32),
        ],
    )
    return f(h144, srcv, dstv, etv, za, zr)


def _tc1_body(relp_ref, emb_ref, wia_ref, wib_ref, bi_ref, wht_ref, bh_ref,
              o_ref):
    rels = relp_ref[0] + relp_ref[1]
    rel_sum = rels[:, :D]
    rel_cnt = rels[:, D:D + 1]
    x_mean = rel_sum / jnp.maximum(rel_cnt, 1.0)
    emb = emb_ref[...]
    gi = (jnp.dot(emb, wia_ref[...], preferred_element_type=jnp.float32)
          + jnp.dot(x_mean, wib_ref[...], preferred_element_type=jnp.float32)
          + bi_ref[...])
    gh = jnp.dot(emb, wht_ref[...], preferred_element_type=jnp.float32) \
        + bh_ref[...]
    i_r, i_z, i_n = gi[:, :D], gi[:, D:2 * D], gi[:, 2 * D:]
    h_r, h_z, h_n = gh[:, :D], gh[:, D:2 * D], gh[:, 2 * D:]
    r = jax.nn.sigmoid(i_r + h_r)
    z = jax.nn.sigmoid(i_z + h_z)
    n = jnp.tanh(i_n + r * h_n)
    h0 = (1.0 - z) * n + z * emb
    rn = _l2norm(h0)
    pad = jnp.zeros((rn.shape[0], DW - D), jnp.float32)
    o_ref[...] = jnp.concatenate([rn, pad], axis=1)


def _tc1(out_r, emb_rel, W_ih, b_ih, W_hh, b_hh):
    emb_p = jnp.zeros((RREL, D), jnp.float32).at[:NR2].set(emb_rel)
    wia = W_ih[:, :D].T
    wib = W_ih[:, D:].T
    wht = W_hh.T
    bi = b_ih.reshape(1, 3 * D)
    bh = b_hh.reshape(1, 3 * D)
    return pl.pallas_call(
        _tc1_body,
        out_shape=jax.ShapeDtypeStruct((RREL, DW), jnp.float32),
    )(out_r, emb_p, wia, wib, bi, wht, bh)


def _sc2_body(t_hbm, dst_hbm, et_hbm, inita_hbm, out_a, dstv, etv, r0, r1,
              semg0, semg1, sema, agg_sh):
    c = lax.axis_index("c")
    s = lax.axis_index("s")
    w = c * NS + s
    pltpu.sync_copy(inita_hbm.at[c].at[pl.ds(s * RROWS, RROWS)],
                    agg_sh.at[pl.ds(s * RROWS, RROWS)])
    plsc.subcore_barrier()
    _edge_pipeline(t_hbm, et_hbm, w, etv,
                   [dst_hbm], [dstv],
                   (r0, r1), (semg0, semg1),
                   [sema], [agg_sh])
    plsc.subcore_barrier()
    pltpu.sync_copy(agg_sh.at[pl.ds(s * RROWS, RROWS)],
                    out_a.at[c].at[pl.ds(s * RROWS, RROWS)])


def _sc2(r_emb_p, dstv, etv, out_a1):
    mesh = plsc.VectorSubcoreMesh(core_axis_name="c", subcore_axis_name="s",
                                  num_cores=NC, num_subcores=NS)
    f = pl.kernel(
        _sc2_body,
        out_type=jax.ShapeDtypeStruct((NC, R, DW), jnp.float32),
        mesh=mesh,
        compiler_params=pltpu.CompilerParams(use_tc_tiling_on_sc=False),
        scratch_types=[
            pltpu.VMEM((CH, B), jnp.int32),
            pltpu.VMEM((CH, B), jnp.int32),
            pltpu.VMEM((B, DW), jnp.float32),
            pltpu.VMEM((B, DW), jnp.float32),
            pltpu.SemaphoreType.DMA,
            pltpu.SemaphoreType.DMA,
            pltpu.SemaphoreType.DMA,
            pltpu.VMEM_SHARED((R, DW), jnp.float32),
        ],
    )
    return f(r_emb_p, dstv, etv, out_a1)


def _tc2_body(agga_ref, h_ref, tw_ref, wn_ref, ws_ref,
              tgw_ref, tgb_ref, fca_ref, fcb_ref, fbias_ref,
              pw_ref, pb_ref, o_ref):
    a144 = agga_ref[0] + agga_ref[1]
    ssum = a144[:, :D]
    deg = a144[:, D:D + 1]
    agg = ssum / jnp.maximum(deg, 1.0)
    h = h_ref[...]
    cur = (jnp.dot(agg, wn_ref[...], preferred_element_type=jnp.float32)
           + jnp.dot(h, ws_ref[...], preferred_element_type=jnp.float32))
    cur = jnp.where(cur >= 0, cur, _SLOPE * cur)
    cur = _l2norm(cur)
    gate = jax.nn.sigmoid(
        jnp.dot(cur, tgw_ref[...], preferred_element_type=jnp.float32)
        + tgb_ref[...])
    h_new = gate * cur + (1.0 - gate) * h
    tw = tw_ref[...]
    xg = jnp.tanh(
        jnp.dot(h_new, fca_ref[...], preferred_element_type=jnp.float32)
        + jnp.dot(tw, fcb_ref[...], preferred_element_type=jnp.float32)
        + fbias_ref[...])
    g = jax.nn.sigmoid(
        jnp.sum(xg * pw_ref[...], axis=-1, keepdims=True) + pb_ref[...])
    o_ref[...] = (1.0 - g) * h_new + g * tw


def _tc2(out_a, h, twin_h, W_neigh, W_self, tgw, tgb,
         fc_W, fc_b, proj_W, proj_b):
    blk = 512
    grid = R // blk
    hp = jnp.zeros((R, D), jnp.float32).at[:N].set(h)
    twp = jnp.zeros((R, D), jnp.float32).at[:N].set(twin_h)
    fca = fc_W[:, :D].T
    fcb = fc_W[:, D:].T
    out = pl.pallas_call(
        _tc2_body,
        grid=(grid,),
        in_specs=[
            pl.BlockSpec((NC, blk, DW), lambda i: (0, i, 0)),
            pl.BlockSpec((blk, D), lambda i: (i, 0)),
            pl.BlockSpec((blk, D), lambda i: (i, 0)),
            pl.BlockSpec((D, D), lambda i: (0, 0)),
            pl.BlockSpec((D, D), lambda i: (0, 0)),
            pl.BlockSpec((D, D), lambda i: (0, 0)),
            pl.BlockSpec((1, D), lambda i: (0, 0)),
            pl.BlockSpec((D, D), lambda i: (0, 0)),
            pl.BlockSpec((D, D), lambda i: (0, 0)),
            pl.BlockSpec((1, D), lambda i: (0, 0)),
            pl.BlockSpec((1, D), lambda i: (0, 0)),
            pl.BlockSpec((1, 1), lambda i: (0, 0)),
        ],
        out_specs=pl.BlockSpec((blk, D), lambda i: (i, 0)),
        out_shape=jax.ShapeDtypeStruct((R, D), jnp.float32),
    )(out_a, hp, twp, W_neigh, W_self, tgw,
      tgb.reshape(1, D), fca, fcb, fc_b.reshape(1, D),
      proj_W.reshape(1, D), proj_b.reshape(1, 1))
    return out[:N]


def kernel(dynamic_emb, emb_rel, W_ih, b_ih, W_hh, b_hh, W_neigh, W_self,
           time_gate_weight, time_gate_bias, fc_W, fc_b, proj_W, proj_b,
           twin_h, edge_index, edge_type):
    pad = EP - E
    src = jnp.concatenate([edge_index[0], jnp.zeros((pad,), jnp.int32)])
    dst = jnp.concatenate([edge_index[1],
                           jnp.full((pad,), N, jnp.int32)])
    et = jnp.concatenate([edge_type, jnp.full((pad,), NR2, jnp.int32)])
    srcv = src.reshape(NW, JB, B)
    dstv = dst.reshape(NW, JB, B)
    etv = et.reshape(NW, JB, B)

    h144 = _tc0(dynamic_emb)
    out_a, out_r = _sc1(h144, srcv, dstv, etv)
    r_emb_p = _tc1(out_r, emb_rel, W_ih, b_ih, W_hh, b_hh)
    out_t = _sc2(r_emb_p, dstv, etv, out_a)
    h = h144[:, :D]
    return _tc2(out_t, h, twin_h, W_neigh, W_self,
                time_gate_weight, time_gate_bias, fc_W, fc_b,
                proj_W, proj_b)


# speedup vs baseline: 6.4298x; 1.1927x over previous
import jax
import jax.numpy as jnp
from jax import lax
from jax.experimental import pallas as pl
from jax.experimental.pallas import tpu as pltpu
from jax.experimental.pallas import tpu_sc as plsc

N = 10000
E = 320000
D = 128
NR2 = 400
DW = 144

NC = 2
NS = 16
NW = NC * NS
B = 112
CH = 6
CH2 = 10
JB = 90
EP = NW * JB * B

R = 10240
RROWS = R // NS
RREL = 416
RRROWS = RREL // NS

_SLOPE = 11.0 / 48.0


def _l2norm(x):
    n = jnp.sqrt(jnp.sum(x * x, axis=-1, keepdims=True))
    return x / jnp.maximum(n, 1e-12)


def _tc0_body(x_ref, o_ref):
    x = x_ref[...]
    hn = _l2norm(x)
    rows = x.shape[0]
    ones = jnp.ones((rows, 1), jnp.float32)
    zeros = jnp.zeros((rows, DW - D - 1), jnp.float32)
    o_ref[...] = jnp.concatenate([hn, ones, zeros], axis=1)


def _tc0(dynamic_emb):
    blk = 400
    return pl.pallas_call(
        _tc0_body,
        grid=(N // blk,),
        in_specs=[pl.BlockSpec((blk, D), lambda i: (i, 0))],
        out_specs=pl.BlockSpec((blk, DW), lambda i: (i, 0)),
        out_shape=jax.ShapeDtypeStruct((N, DW), jnp.float32),
    )(dynamic_emb)


def _edge_pipeline(table_ref, idxsrc_hbm, w, srcv, scat_idx_hbm, scat_idx_v,
                   bufs, semg, scat_sems, scat_dsts, ch):
    def wait_scat():
        for sem, dsh, iv in zip(scat_sems, scat_dsts, scat_idx_v):
            pltpu.make_async_copy(bufs[0], dsh.at[iv.at[0]], sem).wait()

    def chunk(jc, carry):
        @pl.when(jc > 0)
        def _():
            wait_scat()
            wait_scat()
        pltpu.sync_copy(idxsrc_hbm.at[w].at[pl.ds(jc * ch, ch)], srcv)
        for ih, iv in zip(scat_idx_hbm, scat_idx_v):
            pltpu.sync_copy(ih.at[w].at[pl.ds(jc * ch, ch)], iv)
        pltpu.async_copy(table_ref.at[srcv.at[0]], bufs[0], semg[0])
        for j in range(ch):
            buf = bufs[j % 2]
            if j + 1 < ch:
                if j >= 1:
                    wait_scat()
                pltpu.async_copy(table_ref.at[srcv.at[j + 1]],
                                 bufs[(j + 1) % 2], semg[(j + 1) % 2])
            pltpu.make_async_copy(table_ref.at[srcv.at[j]], buf,
                                  semg[j % 2]).wait()
            for sem, dsh, iv in zip(scat_sems, scat_dsts, scat_idx_v):
                pltpu.async_copy(buf, dsh.at[iv.at[j]], sem, add=True)
        return carry

    lax.fori_loop(0, JB // ch, chunk, 0)
    wait_scat()
    wait_scat()


def _sc1_body(h_hbm, src_hbm, dst_hbm, et_hbm, za_hbm, zr_hbm,
              out_a, out_r, srcv, dstv, etv, r0, r1, semg0, semg1,
              sema, semr, aggh_sh, rels_sh):
    c = lax.axis_index("c")
    s = lax.axis_index("s")
    w = c * NS + s
    pltpu.sync_copy(za_hbm, aggh_sh.at[pl.ds(s * RROWS, RROWS)])
    pltpu.sync_copy(zr_hbm, rels_sh.at[pl.ds(s * RRROWS, RRROWS)])
    plsc.subcore_barrier()
    _edge_pipeline(h_hbm, src_hbm, w, srcv,
                   [dst_hbm, et_hbm], [dstv, etv],
                   (r0, r1), (semg0, semg1),
                   [sema, semr], [aggh_sh, rels_sh], CH)
    plsc.subcore_barrier()
    pltpu.sync_copy(aggh_sh.at[pl.ds(s * RROWS, RROWS)],
                    out_a.at[c].at[pl.ds(s * RROWS, RROWS)])
    pltpu.sync_copy(rels_sh.at[pl.ds(s * RRROWS, RRROWS)],
                    out_r.at[c].at[pl.ds(s * RRROWS, RRROWS)])


def _sc1(h144, srcv, dstv, etv):
    mesh = plsc.VectorSubcoreMesh(core_axis_name="c", subcore_axis_name="s",
                                  num_cores=NC, num_subcores=NS)
    za = jnp.zeros((RROWS, DW), jnp.float32)
    zr = jnp.zeros((RRROWS, DW), jnp.float32)
    f = pl.kernel(
        _sc1_body,
        out_type=(jax.ShapeDtypeStruct((NC, R, DW), jnp.float32),
                  jax.ShapeDtypeStruct((NC, RREL, DW), jnp.float32)),
        mesh=mesh,
        compiler_params=pltpu.CompilerParams(use_tc_tiling_on_sc=False),
        scratch_types=[
            pltpu.VMEM((CH, B), jnp.int32),
            pltpu.VMEM((CH, B), jnp.int32),
            pltpu.VMEM((CH, B), jnp.int32),
            pltpu.VMEM((B, DW), jnp.float32),
            pltpu.VMEM((B, DW), jnp.float32),
            pltpu.SemaphoreType.DMA,
            pltpu.SemaphoreType.DMA,
            pltpu.SemaphoreType.DMA,
            pltpu.SemaphoreType.DMA,
            pltpu.VMEM_SHARED((R, DW), jnp.float32),
            pltpu.VMEM_SHARED((RREL, DW), jnp.float32),
        ],
    )
    return f(h144, srcv, dstv, etv, za, zr)


def _tc1_body(relp_ref, emb_ref, wia_ref, wib_ref, bi_ref, wht_ref, bh_ref,
              o_ref):
    rels = relp_ref[0] + relp_ref[1]
    rel_sum = rels[:, :D]
    rel_cnt = rels[:, D:D + 1]
    x_mean = rel_sum / jnp.maximum(rel_cnt, 1.0)
    emb = emb_ref[...]
    gi = (jnp.dot(emb, wia_ref[...], preferred_element_type=jnp.float32)
          + jnp.dot(x_mean, wib_ref[...], preferred_element_type=jnp.float32)
          + bi_ref[...])
    gh = jnp.dot(emb, wht_ref[...], preferred_element_type=jnp.float32) \
        + bh_ref[...]
    i_r, i_z, i_n = gi[:, :D], gi[:, D:2 * D], gi[:, 2 * D:]
    h_r, h_z, h_n = gh[:, :D], gh[:, D:2 * D], gh[:, 2 * D:]
    r = jax.nn.sigmoid(i_r + h_r)
    z = jax.nn.sigmoid(i_z + h_z)
    n = jnp.tanh(i_n + r * h_n)
    h0 = (1.0 - z) * n + z * emb
    rn = _l2norm(h0)
    pad = jnp.zeros((rn.shape[0], DW - D), jnp.float32)
    o_ref[...] = jnp.concatenate([rn, pad], axis=1)


def _tc1(out_r, emb_rel, W_ih, b_ih, W_hh, b_hh):
    emb_p = jnp.zeros((RREL, D), jnp.float32).at[:NR2].set(emb_rel)
    wia = W_ih[:, :D].T
    wib = W_ih[:, D:].T
    wht = W_hh.T
    bi = b_ih.reshape(1, 3 * D)
    bh = b_hh.reshape(1, 3 * D)
    return pl.pallas_call(
        _tc1_body,
        out_shape=jax.ShapeDtypeStruct((RREL, DW), jnp.float32),
    )(out_r, emb_p, wia, wib, bi, wht, bh)


def _sc2_body(t_hbm, dst_hbm, et_hbm, inita_hbm, out_a, dstv, etv, r0, r1,
              semg0, semg1, sema, agg_sh, t_sh):
    c = lax.axis_index("c")
    s = lax.axis_index("s")
    w = c * NS + s
    pltpu.sync_copy(inita_hbm.at[c].at[pl.ds(s * RROWS, RROWS)],
                    agg_sh.at[pl.ds(s * RROWS, RROWS)])
    pltpu.sync_copy(t_hbm.at[pl.ds(s * RRROWS, RRROWS)],
                    t_sh.at[pl.ds(s * RRROWS, RRROWS)])
    plsc.subcore_barrier()
    _edge_pipeline(t_sh, et_hbm, w, etv,
                   [dst_hbm], [dstv],
                   (r0, r1), (semg0, semg1),
                   [sema], [agg_sh], CH2)
    plsc.subcore_barrier()
    pltpu.sync_copy(agg_sh.at[pl.ds(s * RROWS, RROWS)],
                    out_a.at[c].at[pl.ds(s * RROWS, RROWS)])


def _sc2(r_emb_p, dstv, etv, out_a1):
    mesh = plsc.VectorSubcoreMesh(core_axis_name="c", subcore_axis_name="s",
                                  num_cores=NC, num_subcores=NS)
    f = pl.kernel(
        _sc2_body,
        out_type=jax.ShapeDtypeStruct((NC, R, DW), jnp.float32),
        mesh=mesh,
        compiler_params=pltpu.CompilerParams(use_tc_tiling_on_sc=False),
        scratch_types=[
            pltpu.VMEM((CH2, B), jnp.int32),
            pltpu.VMEM((CH2, B), jnp.int32),
            pltpu.VMEM((B, DW), jnp.float32),
            pltpu.VMEM((B, DW), jnp.float32),
            pltpu.SemaphoreType.DMA,
            pltpu.SemaphoreType.DMA,
            pltpu.SemaphoreType.DMA,
            pltpu.VMEM_SHARED((R, DW), jnp.float32),
            pltpu.VMEM_SHARED((RREL, DW), jnp.float32),
        ],
    )
    return f(r_emb_p, dstv, etv, out_a1)


def _tc2_body(agga_ref, h_ref, tw_ref, wn_ref, ws_ref,
              tgw_ref, tgb_ref, fca_ref, fcb_ref, fbias_ref,
              pw_ref, pb_ref, o_ref):
    a144 = agga_ref[0] + agga_ref[1]
    ssum = a144[:, :D]
    deg = a144[:, D:D + 1]
    agg = ssum / jnp.maximum(deg, 1.0)
    h = h_ref[...]
    cur = (jnp.dot(agg, wn_ref[...], preferred_element_type=jnp.float32)
           + jnp.dot(h, ws_ref[...], preferred_element_type=jnp.float32))
    cur = jnp.where(cur >= 0, cur, _SLOPE * cur)
    cur = _l2norm(cur)
    gate = jax.nn.sigmoid(
        jnp.dot(cur, tgw_ref[...], preferred_element_type=jnp.float32)
        + tgb_ref[...])
    h_new = gate * cur + (1.0 - gate) * h
    tw = tw_ref[...]
    xg = jnp.tanh(
        jnp.dot(h_new, fca_ref[...], preferred_element_type=jnp.float32)
        + jnp.dot(tw, fcb_ref[...], preferred_element_type=jnp.float32)
        + fbias_ref[...])
    g = jax.nn.sigmoid(
        jnp.sum(xg * pw_ref[...], axis=-1, keepdims=True) + pb_ref[...])
    o_ref[...] = (1.0 - g) * h_new + g * tw


def _tc2(out_a, h, twin_h, W_neigh, W_self, tgw, tgb,
         fc_W, fc_b, proj_W, proj_b):
    blk = 512
    grid = R // blk
    hp = jnp.zeros((R, D), jnp.float32).at[:N].set(h)
    twp = jnp.zeros((R, D), jnp.float32).at[:N].set(twin_h)
    fca = fc_W[:, :D].T
    fcb = fc_W[:, D:].T
    out = pl.pallas_call(
        _tc2_body,
        grid=(grid,),
        in_specs=[
            pl.BlockSpec((NC, blk, DW), lambda i: (0, i, 0)),
            pl.BlockSpec((blk, D), lambda i: (i, 0)),
            pl.BlockSpec((blk, D), lambda i: (i, 0)),
            pl.BlockSpec((D, D), lambda i: (0, 0)),
            pl.BlockSpec((D, D), lambda i: (0, 0)),
            pl.BlockSpec((D, D), lambda i: (0, 0)),
            pl.BlockSpec((1, D), lambda i: (0, 0)),
            pl.BlockSpec((D, D), lambda i: (0, 0)),
            pl.BlockSpec((D, D), lambda i: (0, 0)),
            pl.BlockSpec((1, D), lambda i: (0, 0)),
            pl.BlockSpec((1, D), lambda i: (0, 0)),
            pl.BlockSpec((1, 1), lambda i: (0, 0)),
        ],
        out_specs=pl.BlockSpec((blk, D), lambda i: (i, 0)),
        out_shape=jax.ShapeDtypeStruct((R, D), jnp.float32),
    )(out_a, hp, twp, W_neigh, W_self, tgw,
      tgb.reshape(1, D), fca, fcb, fc_b.reshape(1, D),
      proj_W.reshape(1, D), proj_b.reshape(1, 1))
    return out[:N]


def kernel(dynamic_emb, emb_rel, W_ih, b_ih, W_hh, b_hh, W_neigh, W_self,
           time_gate_weight, time_gate_bias, fc_W, fc_b, proj_W, proj_b,
           twin_h, edge_index, edge_type):
    pad = EP - E
    src = jnp.concatenate([edge_index[0], jnp.zeros((pad,), jnp.int32)])
    dst = jnp.concatenate([edge_index[1],
                           jnp.full((pad,), N, jnp.int32)])
    et = jnp.concatenate([edge_type, jnp.full((pad,), NR2, jnp.int32)])
    srcv = src.reshape(NW, JB, B)
    dstv = dst.reshape(NW, JB, B)
    etv = et.reshape(NW, JB, B)

    h144 = _tc0(dynamic_emb)
    out_a, out_r = _sc1(h144, srcv, dstv, etv)
    r_emb_p = _tc1(out_r, emb_rel, W_ih, b_ih, W_hh, b_hh)
    out_t = _sc2(r_emb_p, dstv, etv, out_a)
    h = h144[:, :D]
    return _tc2(out_t, h, twin_h, W_neigh, W_self,
                time_gate_weight, time_gate_bias, fc_W, fc_b,
                proj_W, proj_b)
